# Initial kernel scaffold; baseline (speedup 1.0000x reference)
import jax
import jax.numpy as jnp
from jax.experimental import pallas as pl


def kernel(x, edge_index, mask):
    raise NotImplementedError("write your pallas kernel here")


# speedup vs baseline: 4.5524x; 4.5524x over previous
import functools

import jax
import jax.numpy as jnp
from jax import lax
from jax.experimental import pallas as pl
from jax.experimental.pallas import tpu as pltpu
from jax.experimental.pallas import tpu_sc as plsc

NC = 2
NS = 16
CH = 128


@functools.partial(jax.jit, static_argnums=(4, 5, 6))
def _segment_sum_sc(xt, src2, dst2, zer, n_pad, dh, nb):
    mesh = plsc.VectorSubcoreMesh(core_axis_name="c", subcore_axis_name="s")
    rpt = n_pad // NS

    @functools.partial(
        pl.kernel,
        out_type=jax.ShapeDtypeStruct((NC, n_pad, dh), jnp.float32),
        mesh=mesh,
        compiler_params=pltpu.CompilerParams(use_tc_tiling_on_sc=False),
        scratch_types=[
            pltpu.VMEM((CH,), jnp.int32),
            pltpu.VMEM((CH,), jnp.int32),
            pltpu.VMEM((CH, dh), jnp.float32),
            pltpu.VMEM((rpt, dh), jnp.float32),
            pltpu.VMEM_SHARED((n_pad, dh), jnp.float32),
            pltpu.SemaphoreType.DMA,
        ],
    )
    def scatter_kernel(xt_hbm, src_hbm, dst_hbm, zer_hbm, out_hbm,
                       idx_s, idx_d, rows, cop, acc, sem):
        c = lax.axis_index("c")
        s = lax.axis_index("s")

        pltpu.sync_copy(zer_hbm.at[pl.ds(s * rpt, rpt)], cop)
        pltpu.sync_copy(cop, acc.at[pl.ds(s * rpt, rpt)])
        plsc.subcore_barrier()

        niter = (nb - s + NS - 1) // NS

        def body(i, carry):
            j = s + i * NS
            pltpu.sync_copy(src_hbm.at[j], idx_s)
            pltpu.sync_copy(dst_hbm.at[j], idx_d)
            pltpu.async_copy(xt_hbm.at[c].at[idx_s], rows, sem).wait()
            pltpu.sync_copy(rows, acc.at[idx_d], add=True)
            return carry

        lax.fori_loop(0, niter, body, 0)
        plsc.subcore_barrier()

        pltpu.sync_copy(acc.at[pl.ds(s * rpt, rpt)], cop)
        pltpu.sync_copy(cop, out_hbm.at[c].at[pl.ds(s * rpt, rpt)])

    return scatter_kernel(xt, src2, dst2, zer)


def kernel(x, edge_index, mask):
    n, d = x.shape
    e = edge_index.shape[1]
    dh = d // NC
    nb = e // CH
    n_pad = ((n + 8 * NS - 1) // (8 * NS)) * (8 * NS)
    xt = jnp.stack([x[:, :dh], x[:, dh:]], axis=0)
    src2 = edge_index[0].reshape(nb, CH)
    dst2 = edge_index[1].reshape(nb, CH)
    zer = jnp.zeros((n_pad, dh), jnp.float32)
    out2 = _segment_sum_sc(xt, src2, dst2, zer, n_pad, dh, nb)
    return jnp.concatenate([out2[0, :n], out2[1, :n]], axis=1)


# speedup vs baseline: 7.1234x; 1.5648x over previous
import functools

import jax
import jax.numpy as jnp
from jax import lax
from jax.experimental import pallas as pl
from jax.experimental.pallas import tpu as pltpu
from jax.experimental.pallas import tpu_sc as plsc

NC = 2
NS = 16
CH = 128


@functools.partial(jax.jit, static_argnums=(4, 5, 6))
def _segment_sum_sc(xt, src2, dst2, zer, n_pad, dh, nb):
    mesh = plsc.VectorSubcoreMesh(core_axis_name="c", subcore_axis_name="s")
    rpt = n_pad // NS
    g = nb // NS
    npair = g // 2

    @functools.partial(
        pl.kernel,
        out_type=jax.ShapeDtypeStruct((NC, n_pad, dh), jnp.float32),
        mesh=mesh,
        compiler_params=pltpu.CompilerParams(use_tc_tiling_on_sc=False),
        scratch_types=[
            pltpu.VMEM((g, CH), jnp.int32),
            pltpu.VMEM((g, CH), jnp.int32),
            pltpu.VMEM((CH, dh), jnp.float32),
            pltpu.VMEM((CH, dh), jnp.float32),
            pltpu.VMEM_SHARED((n_pad, dh), jnp.float32),
            pltpu.SemaphoreType.DMA,
            pltpu.SemaphoreType.DMA,
        ],
    )
    def scatter_kernel(xt_hbm, src_hbm, dst_hbm, zer_hbm, out_hbm,
                       idx_s, idx_d, r0, r1, acc, sem0, sem1):
        c = lax.axis_index("c")
        s = lax.axis_index("s")

        pltpu.sync_copy(src_hbm.at[pl.ds(s * g, g)], idx_s)
        pltpu.sync_copy(dst_hbm.at[pl.ds(s * g, g)], idx_d)
        pltpu.sync_copy(zer_hbm.at[pl.ds(s * rpt, rpt)], acc.at[pl.ds(s * rpt, rpt)])
        plsc.subcore_barrier()

        xh = xt_hbm.at[c]
        pltpu.async_copy(xh.at[idx_s.at[0]], r0, sem0)
        pltpu.async_copy(xh.at[idx_s.at[1]], r1, sem1)

        def body(p, carry):
            k0 = 2 * p

            pltpu.make_async_copy(xh.at[idx_s.at[k0]], r0, sem0).wait()
            pltpu.sync_copy(r0, acc.at[idx_d.at[k0]], add=True)

            @pl.when(p + 1 < npair)
            def _():
                pltpu.async_copy(xh.at[idx_s.at[k0 + 2]], r0, sem0)

            pltpu.make_async_copy(xh.at[idx_s.at[k0 + 1]], r1, sem1).wait()
            pltpu.sync_copy(r1, acc.at[idx_d.at[k0 + 1]], add=True)

            @pl.when(p + 1 < npair)
            def _():
                pltpu.async_copy(xh.at[idx_s.at[k0 + 3]], r1, sem1)

            return carry

        lax.fori_loop(0, npair, body, 0)
        plsc.subcore_barrier()

        pltpu.sync_copy(acc.at[pl.ds(s * rpt, rpt)],
                        out_hbm.at[c].at[pl.ds(s * rpt, rpt)])

    return scatter_kernel(xt, src2, dst2, zer)


def kernel(x, edge_index, mask):
    n, d = x.shape
    e = edge_index.shape[1]
    dh = d // NC
    n_pad = ((n + 8 * NS - 1) // (8 * NS)) * (8 * NS)
    if n_pad == n:
        n_pad += 8 * NS
    epg = 2 * CH * NS
    e_pad = ((e + epg - 1) // epg) * epg
    nb = e_pad // CH
    src = edge_index[0]
    dst = edge_index[1]
    if e_pad > e:
        pad = e_pad - e
        src = jnp.concatenate([src, jnp.zeros((pad,), jnp.int32)])
        dst = jnp.concatenate([dst, jnp.full((pad,), n, jnp.int32)])
    xt = jnp.stack([x[:, :dh], x[:, dh:]], axis=0)
    src2 = src.reshape(nb, CH)
    dst2 = dst.reshape(nb, CH)
    zer = jnp.zeros((n_pad, dh), jnp.float32)
    out2 = _segment_sum_sc(xt, src2, dst2, zer, n_pad, dh, nb)
    return jnp.concatenate([out2[0, :n], out2[1, :n]], axis=1)
